# baseline (speedup 1.0000x reference)
import jax
import jax.numpy as jnp
from jax.experimental import pallas as pl
from jax.experimental.pallas import tpu as pltpu

_NUM_CLASSES = 20
_SUBLANE = 8


def _round_up(a, m):
    return (a + m - 1) // m * m


def _head_body(x_ref, w_ref, b_ref, o_ref):
    acc = jnp.dot(x_ref[...], w_ref[...], preferred_element_type=jnp.float32)
    acc = acc + b_ref[...]
    acc = jnp.maximum(acc, 0.0)
    o_ref[...] = acc[:, :_NUM_CLASSES]


def kernel(pooled_output, w_t_pad, b_pad):
    n, h = pooled_output.shape
    l_pad = w_t_pad.shape[1]

    tm = min(1024, _round_up(n, _SUBLANE))
    n_pad = _round_up(n, tm)
    x = pooled_output
    if n_pad > n:
        x = jnp.pad(x, ((0, n_pad - n), (0, 0)))

    out = pl.pallas_call(
        _head_body,
        out_shape=jax.ShapeDtypeStruct((n_pad, _NUM_CLASSES), jnp.float32),
        grid=(n_pad // tm,),
        in_specs=[
            pl.BlockSpec((tm, h), lambda i: (i, 0)),
            pl.BlockSpec((h, l_pad), lambda i: (0, 0)),
            pl.BlockSpec((1, l_pad), lambda i: (0, 0)),
        ],
        out_specs=pl.BlockSpec((tm, _NUM_CLASSES), lambda i: (i, 0)),
        compiler_params=pltpu.CompilerParams(
            dimension_semantics=("parallel",),
        ),
    )(x, w_t_pad, b_pad)

    return out[:n]


# speedup vs baseline: 1.0827x; 1.0827x over previous
import jax
import jax.numpy as jnp
from jax.experimental import pallas as pl
from jax.experimental.pallas import tpu as pltpu

_NUM_CLASSES = 20
_SUBLANE = 8


def _round_up(a, m):
    return (a + m - 1) // m * m


def _head_body(x_ref, w_ref, b_ref, o_ref):
    x = x_ref[...].astype(jnp.bfloat16)
    acc = jnp.dot(x, w_ref[...], preferred_element_type=jnp.float32)
    acc = acc + b_ref[...]
    acc = jnp.maximum(acc, 0.0)
    o_ref[...] = acc[:, :_NUM_CLASSES]


def kernel(pooled_output, w_t_pad, b_pad):
    n, h = pooled_output.shape
    l_pad = w_t_pad.shape[1]

    tm = min(2048, _round_up(n, _SUBLANE))
    n_pad = _round_up(n, tm)
    x = pooled_output
    if n_pad > n:
        x = jnp.pad(x, ((0, n_pad - n), (0, 0)))

    out = pl.pallas_call(
        _head_body,
        out_shape=jax.ShapeDtypeStruct((n_pad, _NUM_CLASSES), jnp.float32),
        grid=(n_pad // tm,),
        in_specs=[
            pl.BlockSpec((tm, h), lambda i: (i, 0)),
            pl.BlockSpec((h, l_pad), lambda i: (0, 0)),
            pl.BlockSpec((1, l_pad), lambda i: (0, 0)),
        ],
        out_specs=pl.BlockSpec((tm, _NUM_CLASSES), lambda i: (i, 0)),
        compiler_params=pltpu.CompilerParams(
            dimension_semantics=("parallel",),
        ),
    )(x, w_t_pad.astype(jnp.bfloat16), b_pad)

    return out[:n]


# speedup vs baseline: 1.0899x; 1.0067x over previous
import jax
import jax.numpy as jnp
from jax.experimental import pallas as pl
from jax.experimental.pallas import tpu as pltpu

_NUM_CLASSES = 20
_SUBLANE = 8
_TM = 1024
_NBUF = 4


def _round_up(a, m):
    return (a + m - 1) // m * m


def _make_body(n_chunks):
    def _body(x_hbm, w_ref, b_ref, o_ref, *scratch):
        bufs = scratch[:_NBUF]
        sems = scratch[_NBUF:]

        def copy(i):
            return pltpu.make_async_copy(
                x_hbm.at[pl.ds(i * _TM, _TM), :], bufs[i % _NBUF], sems[i % _NBUF]
            )

        for i in range(min(_NBUF, n_chunks)):
            copy(i).start()

        w = w_ref[...]
        b = b_ref[...]
        for i in range(n_chunks):
            copy(i).wait()
            acc = jnp.dot(bufs[i % _NBUF][...], w,
                          preferred_element_type=jnp.float32)
            acc = jnp.maximum(acc + b, 0.0)
            o_ref[pl.ds(i * _TM, _TM), :] = acc[:, :_NUM_CLASSES]
            nxt = i + _NBUF
            if nxt < n_chunks:
                copy(nxt).start()

    return _body


def kernel(pooled_output, w_t_pad, b_pad):
    n, h = pooled_output.shape

    n_pad = _round_up(n, _TM)
    x = pooled_output
    if n_pad > n:
        x = jnp.pad(x, ((0, n_pad - n), (0, 0)))
    n_chunks = n_pad // _TM

    out = pl.pallas_call(
        _make_body(n_chunks),
        out_shape=jax.ShapeDtypeStruct((n_pad, _NUM_CLASSES), jnp.float32),
        in_specs=[
            pl.BlockSpec(memory_space=pl.ANY),
            pl.BlockSpec(memory_space=pltpu.VMEM),
            pl.BlockSpec(memory_space=pltpu.VMEM),
        ],
        out_specs=pl.BlockSpec(memory_space=pltpu.VMEM),
        scratch_shapes=(
            [pltpu.VMEM((_TM, h), jnp.float32) for _ in range(_NBUF)]
            + [pltpu.SemaphoreType.DMA for _ in range(_NBUF)]
        ),
    )(x, w_t_pad, b_pad)

    return out[:n]


# speedup vs baseline: 1.1496x; 1.0548x over previous
import jax
import jax.numpy as jnp
from jax.experimental import pallas as pl
from jax.experimental.pallas import tpu as pltpu

_NUM_CLASSES = 20
_SUBLANE = 8


def _round_up(a, m):
    return (a + m - 1) // m * m


def _head_body(x1_ref, x2_ref, w1_ref, w2_ref, b_ref, o_ref):
    acc = jnp.dot(x1_ref[...], w1_ref[...], preferred_element_type=jnp.float32)
    acc = acc + jnp.dot(x2_ref[...], w2_ref[...],
                        preferred_element_type=jnp.float32)
    acc = acc + b_ref[...]
    acc = jnp.maximum(acc, 0.0)
    o_ref[...] = acc[:, :_NUM_CLASSES]


def kernel(pooled_output, w_t_pad, b_pad):
    n, h = pooled_output.shape
    l_pad = w_t_pad.shape[1]
    hh = h // 2

    tm = min(2048, _round_up(n, _SUBLANE))
    n_pad = _round_up(n, tm)
    x = pooled_output
    if n_pad > n:
        x = jnp.pad(x, ((0, n_pad - n), (0, 0)))

    out = pl.pallas_call(
        _head_body,
        out_shape=jax.ShapeDtypeStruct((n_pad, _NUM_CLASSES), jnp.float32),
        grid=(n_pad // tm,),
        in_specs=[
            pl.BlockSpec((tm, hh), lambda i: (i, 0)),
            pl.BlockSpec((tm, hh), lambda i: (i, 1)),
            pl.BlockSpec((hh, l_pad), lambda i: (0, 0)),
            pl.BlockSpec((hh, l_pad), lambda i: (1, 0)),
            pl.BlockSpec((1, l_pad), lambda i: (0, 0)),
        ],
        out_specs=pl.BlockSpec((tm, _NUM_CLASSES), lambda i: (i, 0)),
        compiler_params=pltpu.CompilerParams(
            dimension_semantics=("parallel",),
        ),
    )(x, x, w_t_pad, w_t_pad, b_pad)

    return out[:n]
